# baseline (speedup 1.0000x reference)
import jax
import jax.numpy as jnp
from jax.experimental import pallas as pl
from jax.experimental.pallas import tpu as pltpu


def _gather_body(idx_ref, emb_ref, x_ref):
    x_ref[...] = emb_ref[...]


def _matmul_body(x_ref, w_ref, b_ref, out_ref):
    out_ref[...] = jax.lax.dot_general(
        x_ref[...], w_ref[...],
        dimension_numbers=(((1,), (1,)), ((), ())),
        preferred_element_type=jnp.float32,
    ) + b_ref[...]


def kernel(trg, enc_src, trg_mask, src_mask, emb_table, W, b):
    Bb, L = trg.shape
    V, D = emb_table.shape
    idx = trg.reshape(-1).astype(jnp.int32)
    n = idx.shape[0]

    x = pl.pallas_call(
        _gather_body,
        grid_spec=pltpu.PrefetchScalarGridSpec(
            num_scalar_prefetch=1,
            grid=(n,),
            in_specs=[pl.BlockSpec((1, 1, D),
                                   lambda i, idx_ref: (idx_ref[i], 0, 0))],
            out_specs=pl.BlockSpec((1, 1, D), lambda i, idx_ref: (i, 0, 0)),
        ),
        out_shape=jax.ShapeDtypeStruct((n, 1, D), jnp.float32),
    )(idx, emb_table.reshape(V, 1, D)).reshape(n, D)

    BV = 8192
    nv = pl.cdiv(V, BV)
    out = pl.pallas_call(
        _matmul_body,
        grid=(nv,),
        in_specs=[
            pl.BlockSpec((n, D), lambda j: (0, 0)),
            pl.BlockSpec((BV, D), lambda j: (j, 0)),
            pl.BlockSpec((1, BV), lambda j: (0, j)),
        ],
        out_specs=pl.BlockSpec((n, BV), lambda j: (0, j)),
        out_shape=jax.ShapeDtypeStruct((n, V), jnp.float32),
        compiler_params=pltpu.CompilerParams(
            dimension_semantics=("arbitrary",),
        ),
    )(x, W, b.reshape(1, V))
    return out.reshape(Bb, L, V)
